# baseline (speedup 1.0000x reference)
import functools
import math

import jax
import jax.numpy as jnp
from jax import lax
from jax.experimental import pallas as pl
from jax.experimental.pallas import tpu as pltpu
from jax.experimental.pallas import tpu_sc as plsc

F0_BIN = 256
F0_MEL_MIN = 1127.0 * math.log(1.0 + 50.0 / 700.0)
F0_MEL_MAX = 1127.0 * math.log(1.0 + 1100.0 / 700.0)


def _index_body(t_ph, mel2ph_ref, f0_ref, uv_ref, energy_ref,
                gidx_ref, pidx_ref, eidx_ref, mask_ref):
    m = mel2ph_ref[...]
    f0 = f0_ref[...]
    uv = uv_ref[...]
    en = energy_ref[...]
    b_iota = lax.broadcasted_iota(jnp.int32, m.shape, 0)
    gidx_ref[...] = b_iota * t_ph + jnp.maximum(m - 1, 0)
    mask_ref[...] = jnp.broadcast_to(
        (m > 0).astype(jnp.float32)[:, :, None], m.shape + (16,))
    f0d = jnp.exp2(f0)
    f0d = jnp.where(uv > 0, 0.0, f0d)
    f0d = jnp.where(m == 0, 0.0, f0d)
    f0_mel = 1127.0 * jnp.log(1.0 + f0d / 700.0)
    f0_mel = jnp.where(
        f0_mel > 0,
        (f0_mel - F0_MEL_MIN) * (F0_BIN - 2) / (F0_MEL_MAX - F0_MEL_MIN) + 1.0,
        f0_mel)
    f0_mel = jnp.where(f0_mel <= 1.0, 1.0, f0_mel)
    f0_mel = jnp.where(f0_mel > F0_BIN - 1, float(F0_BIN - 1), f0_mel)
    pidx_ref[...] = (f0_mel + 0.5).astype(jnp.int32)
    e = jnp.minimum(jnp.floor(en * (256.0 / 4.0)), 255.0)
    eidx_ref[...] = jnp.maximum(e, 0.0).astype(jnp.int32)


def _compute_indices(mel2ph, f0, uv, energy, t_ph):
    B, T = mel2ph.shape
    i32 = jax.ShapeDtypeStruct((B, T), jnp.int32)
    f32x16 = jax.ShapeDtypeStruct((B, T, 16), jnp.float32)
    return pl.pallas_call(
        functools.partial(_index_body, t_ph),
        out_shape=[i32, i32, i32, f32x16],
    )(mel2ph, f0, uv, energy)


@functools.lru_cache(maxsize=None)
def _make_sc_kernel(rows_total, H, T_mel, NW):
    rpw = rows_total // NW
    NB = 64
    nblk = rpw // NB
    nseg = H // 16
    mesh = plsc.VectorSubcoreMesh(core_axis_name="c", subcore_axis_name="s")

    @functools.partial(
        pl.kernel,
        out_type=jax.ShapeDtypeStruct((rows_total, H), jnp.float32),
        mesh=mesh,
        scratch_types=[
            pltpu.VMEM((NB,), jnp.int32),
            pltpu.VMEM((NB,), jnp.int32),
            pltpu.VMEM((NB,), jnp.int32),
            pltpu.VMEM((NB, 16), jnp.float32),
            pltpu.VMEM((NB, H), jnp.float32),
            pltpu.VMEM((NB, H), jnp.float32),
            pltpu.VMEM((NB, H), jnp.float32),
            pltpu.VMEM((16,), jnp.int32),
            pltpu.VMEM((16, H), jnp.float32),
            pltpu.SemaphoreType.DMA,
            pltpu.SemaphoreType.DMA,
            pltpu.SemaphoreType.DMA,
        ],
    )
    def sck(hub_hbm, gidx_hbm, pidx_hbm, eidx_hbm, mask_hbm,
            wp_hbm, we_hbm, wspk_hbm, sid_hbm, out_hbm,
            gbuf, pbuf, ebuf, maskbuf, hubbuf, ppbuf, eebuf,
            sidbuf, svbuf, sem0, sem1, sem2):
        wid = lax.axis_index("s") * 2 + lax.axis_index("c")
        base = wid * rpw
        b = base // T_mel
        pltpu.sync_copy(sid_hbm, sidbuf)
        pltpu.async_copy(wspk_hbm.at[sidbuf], svbuf, sem0).wait()

        def blk(k, carry):
            off = base + k * NB
            pltpu.sync_copy(gidx_hbm.at[pl.ds(off, NB)], gbuf)
            pltpu.sync_copy(pidx_hbm.at[pl.ds(off, NB)], pbuf)
            pltpu.sync_copy(eidx_hbm.at[pl.ds(off, NB)], ebuf)
            pltpu.sync_copy(mask_hbm.at[pl.ds(off, NB)], maskbuf)
            c0 = pltpu.async_copy(hub_hbm.at[gbuf], hubbuf, sem0)
            c1 = pltpu.async_copy(wp_hbm.at[pbuf], ppbuf, sem1)
            c2 = pltpu.async_copy(we_hbm.at[ebuf], eebuf, sem2)
            c0.wait()
            c1.wait()
            c2.wait()

            def row(i, c):
                maskv = maskbuf[i, :]
                for j in range(nseg):
                    sl = pl.ds(j * 16, 16)
                    s = (hubbuf[i, sl] + ppbuf[i, sl] + eebuf[i, sl]
                         + svbuf[b, sl])
                    hubbuf[i, sl] = s * maskv
                return c

            lax.fori_loop(0, NB, row, 0)
            pltpu.sync_copy(hubbuf, out_hbm.at[pl.ds(off, NB)])
            return carry

        lax.fori_loop(0, nblk, blk, 0)

    return sck


def kernel(hubert, mel2ph, spk_embed, f0, uv, energy, W_spk, W_pitch, W_energy):
    B, T_ph, H = hubert.shape
    T_mel = mel2ph.shape[1]
    gidx, pidx, eidx, mask = _compute_indices(mel2ph, f0, uv, energy, T_ph)
    hub2d = hubert.reshape(B * T_ph, H)
    rows_total = B * T_mel
    sck = _make_sc_kernel(rows_total, H, T_mel, 32)
    out = sck(hub2d, gidx.reshape(-1), pidx.reshape(-1), eidx.reshape(-1),
              mask.reshape(rows_total, 16), W_pitch, W_energy, W_spk,
              spk_embed)
    return out.reshape(B, T_mel, H)
